# Initial kernel scaffold; baseline (speedup 1.0000x reference)
import jax
import jax.numpy as jnp
from jax.experimental import pallas as pl


def kernel(x, edge_index, edge_attr, node_W, node_b, edge_W, edge_b, conv_w0, conv_root0, conv_b0, bn_g0, bn_b0, conv_w1, conv_root1, conv_b1, bn_g1, bn_b1, mlp_W1, mlp_b1, mlp_W2, mlp_b2, mlp_W3, mlp_b3):
    raise NotImplementedError("write your pallas kernel here")


# speedup vs baseline: 1.5537x; 1.5537x over previous
import functools

import jax
import jax.numpy as jnp
from jax import lax
from jax.experimental import pallas as pl
from jax.experimental.pallas import tpu as pltpu
from jax.experimental.pallas import tpu_sc as plsc

NC = 2
NS = 16
NW = NC * NS
EPS = 1e-5
K = 128


def _agg_body(nn, epw, nb, rps, num_r, h_hbm, src_hbm, dkey_hbm, z128_hbm,
              s_hbm, acc, src_v, dkey_v, gidx, sidx, rows, sem):
    nn8 = nn + 8
    cid = lax.axis_index("c")
    sid = lax.axis_index("s")
    wid = sid * NC + cid
    base = wid * epw
    pltpu.sync_copy(src_hbm.at[pl.ds(base, epw)], src_v)
    pltpu.sync_copy(dkey_hbm.at[pl.ds(base, epw)], dkey_v)
    row0 = sid * rps
    zb = z128_hbm.shape[0]
    for z in range(rps // zb):
        pltpu.sync_copy(z128_hbm, acc.at[pl.ds(row0 + z * zb, zb)])
    plsc.subcore_barrier()

    def rel_pass(r, carry):
        def batch(b, carry2):
            for j in range(K // 16):
                o = b * K + j * 16
                sv = src_v[pl.ds(o, 16)]
                kv = dkey_v[pl.ds(o, 16)]
                gidx[pl.ds(j * 16, 16)] = sv
                q = kv - r * nn8
                sm = (q >= 0) & (q < nn8)
                sidx[pl.ds(j * 16, 16)] = jnp.where(sm, q, nn)
            pltpu.async_copy(h_hbm.at[gidx], rows, sem).wait()
            pltpu.sync_copy(rows, acc.at[sidx], add=True)
            return carry2

        lax.fori_loop(0, nb, batch, 0)
        plsc.subcore_barrier()
        pltpu.sync_copy(acc.at[pl.ds(row0, rps)],
                        s_hbm.at[cid * num_r + r, pl.ds(row0, rps)])
        plsc.subcore_barrier()
        return carry

    lax.fori_loop(0, num_r, rel_pass, 0)


def _sc_aggregate(h, src, dkey, num_r):
    n, hdim = h.shape
    e_pad = src.shape[0]
    epw = e_pad // NW
    nb = epw // K
    assert n % (NS * 128) == 0
    rps = n // NS
    zb = 128
    mesh = plsc.VectorSubcoreMesh(core_axis_name="c", subcore_axis_name="s",
                                  num_cores=NC, num_subcores=NS)
    kern = pl.kernel(
        functools.partial(_agg_body, n, epw, nb, rps, num_r),
        out_type=jax.ShapeDtypeStruct((NC * num_r, n, hdim), jnp.float32),
        mesh=mesh,
        scratch_types=[
            pltpu.VMEM_SHARED((n + 8, hdim), jnp.float32),
            pltpu.VMEM((epw,), jnp.int32),
            pltpu.VMEM((epw,), jnp.int32),
            pltpu.VMEM((K,), jnp.int32),
            pltpu.VMEM((K,), jnp.int32),
            pltpu.VMEM((K, hdim), jnp.float32),
            pltpu.SemaphoreType.DMA,
        ],
    )
    z128 = jnp.zeros((zb, hdim), jnp.float32)
    s = kern(h, src, dkey, z128)
    return s.reshape(NC, num_r, n, hdim)


def _cnt_body(nn, epw, nb, rps, tab_hbm, rel_hbm, dst_hbm, z128_hbm,
              c_hbm, acc, rel_v, dst_v, gidx, sidx, rows, sem):
    cid = lax.axis_index("c")
    sid = lax.axis_index("s")
    wid = sid * NC + cid
    base = wid * epw
    pltpu.sync_copy(rel_hbm.at[pl.ds(base, epw)], rel_v)
    pltpu.sync_copy(dst_hbm.at[pl.ds(base, epw)], dst_v)
    row0 = sid * rps
    zb = z128_hbm.shape[0]
    for z in range(rps // zb):
        pltpu.sync_copy(z128_hbm, acc.at[pl.ds(row0 + z * zb, zb)])
    plsc.subcore_barrier()

    def batch(b, carry):
        for j in range(K // 16):
            o = b * K + j * 16
            gidx[pl.ds(j * 16, 16)] = rel_v[pl.ds(o, 16)]
            sidx[pl.ds(j * 16, 16)] = dst_v[pl.ds(o, 16)]
        pltpu.async_copy(tab_hbm.at[gidx], rows, sem).wait()
        pltpu.sync_copy(rows, acc.at[sidx], add=True)
        return carry

    lax.fori_loop(0, nb, batch, 0)
    plsc.subcore_barrier()
    pltpu.sync_copy(acc.at[pl.ds(row0, rps)], c_hbm.at[cid, pl.ds(row0, rps)])


def _sc_counts(dst_p, rel_p, n, num_r):
    e_pad = dst_p.shape[0]
    epw = e_pad // NW
    nb = epw // K
    rps = n // NS
    zb = 128
    tab = jnp.concatenate(
        [jnp.eye(num_r, 128, dtype=jnp.float32),
         jnp.zeros((16 - num_r, 128), jnp.float32)], axis=0)
    mesh = plsc.VectorSubcoreMesh(core_axis_name="c", subcore_axis_name="s",
                                  num_cores=NC, num_subcores=NS)
    kern = pl.kernel(
        functools.partial(_cnt_body, n, epw, nb, rps),
        out_type=jax.ShapeDtypeStruct((NC, n, 128), jnp.float32),
        mesh=mesh,
        scratch_types=[
            pltpu.VMEM_SHARED((n + 8, 128), jnp.float32),
            pltpu.VMEM((epw,), jnp.int32),
            pltpu.VMEM((epw,), jnp.int32),
            pltpu.VMEM((K,), jnp.int32),
            pltpu.VMEM((K,), jnp.int32),
            pltpu.VMEM((K, 128), jnp.float32),
            pltpu.SemaphoreType.DMA,
        ],
    )
    z128 = jnp.zeros((zb, 128), jnp.float32)
    return kern(tab, rel_p, dst_p, z128)


def _pair_body(epw, nbg, h_hbm, src_hbm, dst_hbm, hs_hbm, hd_hbm,
               idx_v, rows, sem):
    cid = lax.axis_index("c")
    sid = lax.axis_index("s")
    wid = sid * NC + cid
    base = wid * epw
    for which, (arr, out) in enumerate(((src_hbm, hs_hbm), (dst_hbm, hd_hbm))):
        pltpu.sync_copy(arr.at[pl.ds(base, epw)], idx_v.at[pl.ds(0, epw)])

        def batch(b, carry):
            pltpu.async_copy(h_hbm.at[idx_v.at[pl.ds(b * K, K)]], rows,
                             sem).wait()
            pltpu.sync_copy(rows, out.at[pl.ds(base + b * K, K)])
            return carry

        lax.fori_loop(0, nbg, batch, 0)


def _sc_pair_gather(h, src_pad, dst_pad):
    n, hdim = h.shape
    e_pad = src_pad.shape[0]
    epw = e_pad // NW
    nbg = epw // K
    mesh = plsc.VectorSubcoreMesh(core_axis_name="c", subcore_axis_name="s",
                                  num_cores=NC, num_subcores=NS)
    kern = pl.kernel(
        functools.partial(_pair_body, epw, nbg),
        out_type=(jax.ShapeDtypeStruct((e_pad, hdim), jnp.float32),
                  jax.ShapeDtypeStruct((e_pad, hdim), jnp.float32)),
        mesh=mesh,
        scratch_types=[
            pltpu.VMEM((epw,), jnp.int32),
            pltpu.VMEM((K, hdim), jnp.float32),
            pltpu.SemaphoreType.DMA,
        ],
    )
    return kern(h, src_pad, dst_pad)


def _t1_body(x_ref, w_ref, b_ref, o_ref):
    o_ref[...] = (jnp.dot(x_ref[...], w_ref[...],
                          preferred_element_type=jnp.float32) + b_ref[...])


def _node_linear(x, w, b, bn):
    n, din = x.shape
    h = w.shape[1]
    return pl.pallas_call(
        _t1_body,
        grid=(n // bn,),
        in_specs=[pl.BlockSpec((bn, din), lambda i: (i, 0)),
                  pl.BlockSpec((din, h), lambda i: (0, 0)),
                  pl.BlockSpec((1, h), lambda i: (0, 0))],
        out_specs=pl.BlockSpec((bn, h), lambda i: (i, 0)),
        out_shape=jax.ShapeDtypeStruct((n, h), jnp.float32),
    )(x, w, b[None])


def _t2_body(num_r, n_true, bn, h_ref, s_ref, c_ref, root_ref, w_ref, b_ref,
             acc_ref, mom_ref):
    a = (jnp.dot(h_ref[...], root_ref[...],
                 preferred_element_type=jnp.float32) + b_ref[...])
    s_prev = 0.0
    for r in range(num_r):
        s_cum = s_ref[0, r] + s_ref[1, r]
        s = s_cum - s_prev
        s_prev = s_cum
        c = jnp.maximum(c_ref[0, :, r] + c_ref[1, :, r], 1.0)
        a = a + jnp.dot(s / c[:, None], w_ref[r],
                        preferred_element_type=jnp.float32)
    acc_ref[...] = a
    hdim = a.shape[1]
    ridx = (lax.broadcasted_iota(jnp.int32, (bn, 1), 0)
            + pl.program_id(0) * bn)
    am = jnp.where(ridx < n_true, a, 0.0)
    mom = jnp.concatenate(
        [jnp.sum(am, axis=0)[None], jnp.sum(am * am, axis=0)[None],
         jnp.zeros((6, hdim), jnp.float32)], axis=0)
    mom_ref[...] = mom[None]


def _conv_dense(h, s, c, root, w, b, n_true, bn):
    n, hdim = h.shape
    num_r = w.shape[0]
    g = n // bn
    return pl.pallas_call(
        functools.partial(_t2_body, num_r, n_true, bn),
        grid=(g,),
        in_specs=[pl.BlockSpec((bn, hdim), lambda i: (i, 0)),
                  pl.BlockSpec((NC, num_r, bn, hdim), lambda i: (0, 0, i, 0)),
                  pl.BlockSpec((NC, bn, 128), lambda i: (0, i, 0)),
                  pl.BlockSpec((hdim, hdim), lambda i: (0, 0)),
                  pl.BlockSpec((num_r, hdim, hdim), lambda i: (0, 0, 0)),
                  pl.BlockSpec((1, hdim), lambda i: (0, 0))],
        out_specs=(pl.BlockSpec((bn, hdim), lambda i: (i, 0)),
                   pl.BlockSpec((1, 8, hdim), lambda i: (i, 0, 0))),
        out_shape=(jax.ShapeDtypeStruct((n, hdim), jnp.float32),
                   jax.ShapeDtypeStruct((g, 8, hdim), jnp.float32)),
    )(h, s, c, root, w, b[None])


def _t3_body(n, acc_ref, mom_ref, h_ref, g_ref, b_ref, o_ref):
    m = jnp.sum(mom_ref[:, 0, :], axis=0) / n
    v = jnp.sum(mom_ref[:, 1, :], axis=0) / n - m * m
    y = (acc_ref[...] - m[None]) * lax.rsqrt(v + EPS)[None] * g_ref[...] \
        + b_ref[...]
    o_ref[...] = (h_ref[...] + jnp.maximum(y, 0.0)) / 2.0


def _bn_residual(acc, mom, h, g, b, n_true, bn):
    n, hdim = h.shape
    gg = mom.shape[0]
    return pl.pallas_call(
        functools.partial(_t3_body, float(n_true)),
        grid=(n // bn,),
        in_specs=[pl.BlockSpec((bn, hdim), lambda i: (i, 0)),
                  pl.BlockSpec((gg, 8, hdim), lambda i: (0, 0, 0)),
                  pl.BlockSpec((bn, hdim), lambda i: (i, 0)),
                  pl.BlockSpec((1, hdim), lambda i: (0, 0)),
                  pl.BlockSpec((1, hdim), lambda i: (0, 0))],
        out_specs=pl.BlockSpec((bn, hdim), lambda i: (i, 0)),
        out_shape=jax.ShapeDtypeStruct((n, hdim), jnp.float32),
    )(acc, mom, h, g[None], b[None])


def _t4_body(hs_ref, hd_ref, ea_ref, ew_ref, eb_ref, w1a_ref, w1b_ref,
             w1c_ref, b1_ref, w2_ref, b2_ref, w3_ref, b3_ref, o_ref):
    dot = functools.partial(jnp.dot, preferred_element_type=jnp.float32)
    ea = dot(ea_ref[...], ew_ref[...]) + eb_ref[...]
    z1 = (dot(jnp.maximum(hs_ref[...], 0.0), w1a_ref[...])
          + dot(jnp.maximum(hd_ref[...], 0.0), w1b_ref[...])
          + dot(ea, w1c_ref[...]) + b1_ref[...])
    z1 = jnp.maximum(z1, 0.0)
    z2 = jnp.maximum(dot(z1, w2_ref[...]) + b2_ref[...], 0.0)
    o_ref[...] = dot(z2, w3_ref[...]) + b3_ref[...]


def _edge_mlp(hs, hd, eattr, ew, eb, w1, b1, w2, b2, w3, b3, bm):
    e_pad, hdim = hs.shape
    de = eattr.shape[1]
    d1 = w1.shape[1]
    d2 = w2.shape[1]
    d3 = w3.shape[1]
    w1a = w1[:hdim]
    w1b = w1[hdim:2 * hdim]
    w1c = w1[2 * hdim:]
    return pl.pallas_call(
        _t4_body,
        grid=(e_pad // bm,),
        in_specs=[pl.BlockSpec((bm, hdim), lambda i: (i, 0)),
                  pl.BlockSpec((bm, hdim), lambda i: (i, 0)),
                  pl.BlockSpec((bm, de), lambda i: (i, 0)),
                  pl.BlockSpec((de, hdim), lambda i: (0, 0)),
                  pl.BlockSpec((1, hdim), lambda i: (0, 0)),
                  pl.BlockSpec((hdim, d1), lambda i: (0, 0)),
                  pl.BlockSpec((hdim, d1), lambda i: (0, 0)),
                  pl.BlockSpec((hdim, d1), lambda i: (0, 0)),
                  pl.BlockSpec((1, d1), lambda i: (0, 0)),
                  pl.BlockSpec((d1, d2), lambda i: (0, 0)),
                  pl.BlockSpec((1, d2), lambda i: (0, 0)),
                  pl.BlockSpec((d2, d3), lambda i: (0, 0)),
                  pl.BlockSpec((1, d3), lambda i: (0, 0))],
        out_specs=pl.BlockSpec((bm, d3), lambda i: (i, 0)),
        out_shape=jax.ShapeDtypeStruct((e_pad, d3), jnp.float32),
    )(hs, hd, eattr, ew, eb[None], w1a, w1b, w1c, b1[None], w2, b2[None],
      w3, b3[None])


def kernel(x, edge_index, edge_attr, node_W, node_b, edge_W, edge_b,
           conv_w0, conv_root0, conv_b0, bn_g0, bn_b0,
           conv_w1, conv_root1, conv_b1, bn_g1, bn_b1,
           mlp_W1, mlp_b1, mlp_W2, mlp_b2, mlp_W3, mlp_b3):
    n = x.shape[0]
    n_pad = -(-n // (NS * 128)) * (NS * 128)
    e = edge_index.shape[1]
    num_r = conv_w0.shape[0]
    src = edge_index[0].astype(jnp.int32)
    dst = edge_index[1].astype(jnp.int32)
    rel = edge_attr[:, -1].astype(jnp.int32)

    x_p = jnp.pad(x, ((0, n_pad - n), (0, 0)))
    e_pad = -(-e // (NW * K)) * (NW * K)
    src_p = jnp.pad(src, (0, e_pad - e))
    dst_p = jnp.pad(dst, (0, e_pad - e))
    ea_p = jnp.pad(edge_attr, ((0, e_pad - e), (0, 0)))
    dkey_p = jnp.pad(dst + (n_pad + 8) * rel, (0, e_pad - e),
                     constant_values=-1)
    dst_dummy = jnp.pad(dst, (0, e_pad - e), constant_values=n_pad)
    rel_p = jnp.pad(rel, (0, e_pad - e), constant_values=num_r)

    cnt = _sc_counts(dst_dummy, rel_p, n_pad, num_r)
    h = _node_linear(x_p, node_W, node_b, bn=640)
    for (w, root, b, g, bb) in ((conv_w0, conv_root0, conv_b0, bn_g0, bn_b0),
                                (conv_w1, conv_root1, conv_b1, bn_g1, bn_b1)):
        s = _sc_aggregate(h, src_p, dkey_p, num_r)
        acc, mom = _conv_dense(h, s, cnt, root, w, b, n, bn=640)
        h = _bn_residual(acc, mom, h, g, bb, n, bn=640)
    hs, hd = _sc_pair_gather(h, src_p, dst_p)
    z = _edge_mlp(hs, hd, ea_p, edge_W, edge_b, mlp_W1, mlp_b1, mlp_W2,
                  mlp_b2, mlp_W3, mlp_b3, bm=2048)
    return z[:e]
